# baseline (speedup 1.0000x reference)
import jax
import jax.numpy as jnp
import numpy as np
from jax import lax
from jax.experimental import pallas as pl
from jax.experimental.pallas import tpu as pltpu
from jax.experimental.pallas import tpu_sc as plsc

_RC = 4.6
_FACTOR = 0.5 * 27.211386245988 * 0.529177210903
_NUM_MOLS = 100
_N_ATOMS = 100000
_N_EDGES = 3200000

_NC = 2
_NS = 16
_NW = _NC * _NS

_K = 5
_B = 2000
_EPC = _N_EDGES // _K
_EPT = _EPC // _NW
_NCH = _EPT // _B
_NV = _B // 16


def _compute_chunk(ibufp, jbufp, ri, rj, acc, lanes, laneoff):
    def _one_vreg(off):
        rsel = lanes + off
        col0 = lanes * 0
        iv = ibufp[pl.ds(off, 16)]
        jv = jbufp[pl.ds(off, 16)]
        xi = plsc.load_gather(ri, [rsel, col0])
        yi = plsc.load_gather(ri, [rsel, col0 + 1])
        zi = plsc.load_gather(ri, [rsel, col0 + 2])
        qi = plsc.load_gather(ri, [rsel, col0 + 3])
        molf = plsc.load_gather(ri, [rsel, col0 + 4])
        xj = plsc.load_gather(rj, [rsel, col0])
        yj = plsc.load_gather(rj, [rsel, col0 + 1])
        zj = plsc.load_gather(rj, [rsel, col0 + 2])
        qj = plsc.load_gather(rj, [rsel, col0 + 3])

        dx = xi - xj
        dy = yi - yj
        dz = zi - zj
        r2 = dx * dx + dy * dy + dz * dz + np.float32(1e-12)
        seed = jnp.int32(0x5F3759DF) - (plsc.bitcast(r2, jnp.int32) >> 1)
        y = plsc.bitcast(seed, jnp.float32)
        hr = np.float32(0.5) * r2
        y = y * (np.float32(1.5) - hr * y * y)
        y = y * (np.float32(1.5) - hr * y * y)
        inv_d = y
        t = jnp.maximum(np.float32(1.0) - r2 * np.float32(1.0 / (_RC * _RC)),
                        np.float32(1e-6))
        fc = jnp.exp(np.float32(1.0) - np.float32(1.0) / t)

        mol = molf.astype(jnp.int32)
        e = (np.float32(1.0) - fc) * (qi * qj) * inv_d
        e = jnp.where(iv != jv, e, np.float32(0.0))
        plsc.addupdate_scatter(acc, [laneoff + mol], e)

    def vbody(v, carry):
        for u in range(25):
            _one_vreg(v * jnp.int32(400) + jnp.int32(u * 16))
        return carry

    lax.fori_loop(jnp.int32(0), jnp.int32(_NV // 25), vbody, jnp.int32(0),
                  unroll=False)


def _tile_body(table, eij, out, shared, ibuf, jbuf, rows_i, rows_j, acc,
               obuf, sem_rows, sem_idx):
    sid = lax.axis_index("s")
    wid = sid * _NC + lax.axis_index("c")
    lanes = jnp.arange(16, dtype=jnp.int32)
    laneoff = lanes * jnp.int32(128)
    zero16 = jnp.zeros(16, dtype=jnp.float32)
    base = wid * jnp.int32(_EPT)

    for w in range(128):
        acc[pl.ds(w * 16, 16)] = zero16

    @pl.when(sid == jnp.int32(0))
    def _():
        pltpu.sync_copy(table, shared)

    plsc.subcore_barrier()

    def issue_rows(c, p):
        pltpu.async_copy(shared.at[ibuf.at[p]], rows_i.at[p], sem_rows)
        pltpu.async_copy(shared.at[jbuf.at[p]], rows_j.at[p], sem_rows)

    def wait_rows(p):
        pltpu.make_async_copy(shared.at[ibuf.at[p]], rows_i.at[p],
                              sem_rows).wait()
        pltpu.make_async_copy(shared.at[jbuf.at[p]], rows_j.at[p],
                              sem_rows).wait()

    def issue_idx(c, p):
        off = base + c * jnp.int32(_B)
        pltpu.async_copy(eij.at[jnp.int32(0), pl.ds(off, _B)], ibuf.at[p],
                         sem_idx)
        pltpu.async_copy(eij.at[jnp.int32(1), pl.ds(off, _B)], jbuf.at[p],
                         sem_idx)

    def wait_idx(c, p):
        off = base + c * jnp.int32(_B)
        pltpu.make_async_copy(eij.at[jnp.int32(0), pl.ds(off, _B)],
                              ibuf.at[p], sem_idx).wait()
        pltpu.make_async_copy(eij.at[jnp.int32(1), pl.ds(off, _B)],
                              jbuf.at[p], sem_idx).wait()

    pltpu.sync_copy(eij.at[jnp.int32(0), pl.ds(base, _B)],
                    ibuf.at[jnp.int32(0)])
    pltpu.sync_copy(eij.at[jnp.int32(1), pl.ds(base, _B)],
                    jbuf.at[jnp.int32(0)])
    issue_rows(jnp.int32(0), jnp.int32(0))
    issue_idx(jnp.int32(1), jnp.int32(1))

    def chunk_body(c, carry):
        p = c & jnp.int32(1)
        q = p ^ jnp.int32(1)

        @pl.when(c + jnp.int32(1) < jnp.int32(_NCH))
        def _():
            wait_idx(c + jnp.int32(1), q)
            issue_rows(c + jnp.int32(1), q)

        wait_rows(p)
        _compute_chunk(ibuf.at[p], jbuf.at[p], rows_i.at[p], rows_j.at[p],
                       acc, lanes, laneoff)

        @pl.when(c + jnp.int32(2) < jnp.int32(_NCH))
        def _():
            issue_idx(c + jnp.int32(2), p)

        return carry

    lax.fori_loop(jnp.int32(0), jnp.int32(_NCH), chunk_body, jnp.int32(0),
                  unroll=False)

    for cg in range(8):
        s = acc[pl.ds(cg * 16, 16)]
        for r in range(1, 16):
            s = s + acc[pl.ds(r * 128 + cg * 16, 16)]
        obuf[pl.ds(cg * 16, 16)] = s
    pltpu.sync_copy(obuf, out.at[wid])


@jax.jit
def _lr_coulomb_sc(table, eij):
    mesh = plsc.VectorSubcoreMesh(core_axis_name="c", subcore_axis_name="s")
    f = pl.kernel(
        _tile_body,
        out_type=jax.ShapeDtypeStruct((_NW, 128), jnp.float32),
        mesh=mesh,
        compiler_params=pltpu.CompilerParams(
            needs_layout_passes=False, use_tc_tiling_on_sc=False),
        scratch_types=[
            pltpu.VMEM_SHARED((_N_ATOMS, 8), jnp.float32),
            pltpu.VMEM((2, _B), jnp.int32),
            pltpu.VMEM((2, _B), jnp.int32),
            pltpu.VMEM((2, _B, 8), jnp.float32),
            pltpu.VMEM((2, _B, 8), jnp.float32),
            pltpu.VMEM((2048,), jnp.float32),
            pltpu.VMEM((128,), jnp.float32),
            pltpu.SemaphoreType.DMA,
            pltpu.SemaphoreType.DMA,
        ],
    )
    return f(table, eij)


def kernel(coord, charges, edge_index, mol_idx):
    coord = coord.astype(jnp.float32)
    q = charges.astype(jnp.float32)
    molf = mol_idx.astype(jnp.float32)
    table = jnp.concatenate(
        [coord, q[:, None], molf[:, None],
         jnp.zeros((_N_ATOMS, 3), jnp.float32)], axis=1)

    acc = None
    for k in range(_K):
        eij = edge_index[:, k * _EPC:(k + 1) * _EPC].astype(jnp.int32)
        part = _lr_coulomb_sc(table, eij)
        s = jnp.sum(part.astype(jnp.float64), axis=0)
        acc = s if acc is None else acc + s
    return _FACTOR * acc[:_NUM_MOLS]


# speedup vs baseline: 1.0732x; 1.0732x over previous
import jax
import jax.numpy as jnp
import numpy as np
from jax import lax
from jax.experimental import pallas as pl
from jax.experimental.pallas import tpu as pltpu
from jax.experimental.pallas import tpu_sc as plsc

_RC = 4.6
_FACTOR = 0.5 * 27.211386245988 * 0.529177210903
_NUM_MOLS = 100
_N_ATOMS = 100000
_N_EDGES = 3200000

_NC = 2
_NS = 16
_NW = _NC * _NS

_B = 2000
_EPT = _N_EDGES // _NW
_NCH = _EPT // _B
_NV = _B // 16


def _compute_chunk(ibufp, jbufp, ri, rj, acc, lanes, laneoff):
    def _one_vreg(off):
        rsel = lanes + off
        col0 = lanes * 0
        iv = ibufp[pl.ds(off, 16)]
        jv = jbufp[pl.ds(off, 16)]
        xi = plsc.load_gather(ri, [rsel, col0])
        yi = plsc.load_gather(ri, [rsel, col0 + 1])
        zi = plsc.load_gather(ri, [rsel, col0 + 2])
        qi = plsc.load_gather(ri, [rsel, col0 + 3])
        molf = plsc.load_gather(ri, [rsel, col0 + 4])
        xj = plsc.load_gather(rj, [rsel, col0])
        yj = plsc.load_gather(rj, [rsel, col0 + 1])
        zj = plsc.load_gather(rj, [rsel, col0 + 2])
        qj = plsc.load_gather(rj, [rsel, col0 + 3])

        dx = xi - xj
        dy = yi - yj
        dz = zi - zj
        r2 = dx * dx + dy * dy + dz * dz + np.float32(1e-12)
        seed = jnp.int32(0x5F3759DF) - (plsc.bitcast(r2, jnp.int32) >> 1)
        y = plsc.bitcast(seed, jnp.float32)
        hr = np.float32(0.5) * r2
        y = y * (np.float32(1.5) - hr * y * y)
        y = y * (np.float32(1.5) - hr * y * y)
        inv_d = y
        t = jnp.maximum(np.float32(1.0) - r2 * np.float32(1.0 / (_RC * _RC)),
                        np.float32(1e-6))
        fc = jnp.exp(np.float32(1.0) - np.float32(1.0) / t)

        mol = molf.astype(jnp.int32)
        e = (np.float32(1.0) - fc) * (qi * qj) * inv_d
        e = jnp.where(iv != jv, e, np.float32(0.0))
        plsc.addupdate_scatter(acc, [laneoff + mol], e)

    def vbody(v, carry):
        for u in range(25):
            _one_vreg(v * jnp.int32(400) + jnp.int32(u * 16))
        return carry

    lax.fori_loop(jnp.int32(0), jnp.int32(_NV // 25), vbody, jnp.int32(0),
                  unroll=False)


def _tile_body(table, eij, out, shared, ibuf, jbuf, rows_i, rows_j, acc,
               obuf, sem_rows, sem_idx):
    sid = lax.axis_index("s")
    wid = sid * _NC + lax.axis_index("c")
    lanes = jnp.arange(16, dtype=jnp.int32)
    laneoff = lanes * jnp.int32(128)
    zero16 = jnp.zeros(16, dtype=jnp.float32)
    base = wid * jnp.int32(_EPT)

    for w in range(128):
        acc[pl.ds(w * 16, 16)] = zero16

    @pl.when(sid == jnp.int32(0))
    def _():
        pltpu.sync_copy(table, shared)

    plsc.subcore_barrier()

    def issue_rows(c, p):
        pltpu.async_copy(shared.at[ibuf.at[p]], rows_i.at[p], sem_rows)
        pltpu.async_copy(shared.at[jbuf.at[p]], rows_j.at[p], sem_rows)

    def wait_rows(p):
        pltpu.make_async_copy(shared.at[ibuf.at[p]], rows_i.at[p],
                              sem_rows).wait()
        pltpu.make_async_copy(shared.at[jbuf.at[p]], rows_j.at[p],
                              sem_rows).wait()

    def issue_idx(c, p):
        off = base + c * jnp.int32(_B)
        pltpu.async_copy(eij.at[jnp.int32(0), pl.ds(off, _B)], ibuf.at[p],
                         sem_idx)
        pltpu.async_copy(eij.at[jnp.int32(1), pl.ds(off, _B)], jbuf.at[p],
                         sem_idx)

    def wait_idx(c, p):
        off = base + c * jnp.int32(_B)
        pltpu.make_async_copy(eij.at[jnp.int32(0), pl.ds(off, _B)],
                              ibuf.at[p], sem_idx).wait()
        pltpu.make_async_copy(eij.at[jnp.int32(1), pl.ds(off, _B)],
                              jbuf.at[p], sem_idx).wait()

    pltpu.sync_copy(eij.at[jnp.int32(0), pl.ds(base, _B)],
                    ibuf.at[jnp.int32(0)])
    pltpu.sync_copy(eij.at[jnp.int32(1), pl.ds(base, _B)],
                    jbuf.at[jnp.int32(0)])
    issue_rows(jnp.int32(0), jnp.int32(0))
    issue_idx(jnp.int32(1), jnp.int32(1))

    def chunk_body(c, carry):
        p = c & jnp.int32(1)
        q = p ^ jnp.int32(1)

        @pl.when(c + jnp.int32(1) < jnp.int32(_NCH))
        def _():
            wait_idx(c + jnp.int32(1), q)
            issue_rows(c + jnp.int32(1), q)

        wait_rows(p)
        _compute_chunk(ibuf.at[p], jbuf.at[p], rows_i.at[p], rows_j.at[p],
                       acc, lanes, laneoff)

        @pl.when(c + jnp.int32(2) < jnp.int32(_NCH))
        def _():
            issue_idx(c + jnp.int32(2), p)

        return carry

    lax.fori_loop(jnp.int32(0), jnp.int32(_NCH), chunk_body, jnp.int32(0),
                  unroll=False)

    for cg in range(8):
        s = acc[pl.ds(cg * 16, 16)]
        for r in range(1, 16):
            s = s + acc[pl.ds(r * 128 + cg * 16, 16)]
        obuf[pl.ds(cg * 16, 16)] = s
    pltpu.sync_copy(obuf, out.at[wid])


@jax.jit
def _lr_coulomb_sc(table, eij):
    mesh = plsc.VectorSubcoreMesh(core_axis_name="c", subcore_axis_name="s")
    f = pl.kernel(
        _tile_body,
        out_type=jax.ShapeDtypeStruct((_NW, 128), jnp.float32),
        mesh=mesh,
        compiler_params=pltpu.CompilerParams(
            needs_layout_passes=False, use_tc_tiling_on_sc=False),
        scratch_types=[
            pltpu.VMEM_SHARED((_N_ATOMS, 8), jnp.float32),
            pltpu.VMEM((2, _B), jnp.int32),
            pltpu.VMEM((2, _B), jnp.int32),
            pltpu.VMEM((2, _B, 8), jnp.float32),
            pltpu.VMEM((2, _B, 8), jnp.float32),
            pltpu.VMEM((2048,), jnp.float32),
            pltpu.VMEM((128,), jnp.float32),
            pltpu.SemaphoreType.DMA,
            pltpu.SemaphoreType.DMA,
        ],
    )
    return f(table, eij)


def kernel(coord, charges, edge_index, mol_idx):
    coord = coord.astype(jnp.float32)
    q = charges.astype(jnp.float32)
    molf = mol_idx.astype(jnp.float32)
    table = jnp.concatenate(
        [coord, q[:, None], molf[:, None],
         jnp.zeros((_N_ATOMS, 3), jnp.float32)], axis=1)

    eij = edge_index.astype(jnp.int32)

    partials = _lr_coulomb_sc(table, eij)
    e_mol = jnp.sum(partials.astype(jnp.float64), axis=0)[:_NUM_MOLS]
    return _FACTOR * e_mol
